# Initial kernel scaffold; baseline (speedup 1.0000x reference)
import jax
import jax.numpy as jnp
from jax.experimental import pallas as pl


def kernel(x, edge_index, batch, W1, b1, W2, b2, Wlin, blin):
    raise NotImplementedError("write your pallas kernel here")


# speedup vs baseline: 28.8963x; 28.8963x over previous
import jax
import jax.numpy as jnp
from jax import lax
from jax.experimental import pallas as pl
from jax.experimental.pallas import tpu as pltpu
from jax.experimental.pallas import tpu_sc as plsc

N = 100000
G = 256
NC, NS = 2, 16
NW = NC * NS
NPAD = 100352
NPT = NPAD // NS
K = 2048
NB = 100
TPB = K * NB
EPAD = TPB * NW
NGP = 384

_mesh = plsc.VectorSubcoreMesh(
    core_axis_name="c", subcore_axis_name="s", num_cores=NC, num_subcores=NS)


def _wid():
    return lax.axis_index("s") * NC + lax.axis_index("c")


def _p1_body(d2_ref, batp_ref, ones_ref, zeros_ref,
             deg_out, ng_out, degT, ngT, oneb, didx, bidx):
    c = lax.axis_index("c")
    s = lax.axis_index("s")
    wid = _wid()
    pltpu.sync_copy(ones_ref, oneb)
    pltpu.sync_copy(zeros_ref.at[pl.ds(s * NPT, NPT)],
                    degT.at[pl.ds(s * NPT, NPT)])

    @pl.when(jnp.logical_and(c == 0, s == 0))
    def _():
        pltpu.sync_copy(zeros_ref.at[pl.ds(0, NGP)], ngT)
    plsc.subcore_barrier()

    def eblk(i, carry):
        base = wid * TPB + i * K
        pltpu.sync_copy(d2_ref.at[pl.ds(base, K)], didx)
        pltpu.sync_copy(oneb.at[pl.ds(0, K)], degT.at[didx], add=True)
        return carry

    lax.fori_loop(0, NB, eblk, 0)

    @pl.when(c == 0)
    def _():
        pltpu.sync_copy(batp_ref.at[pl.ds(s * NPT, NPT)], bidx)
        pltpu.sync_copy(oneb, ngT.at[bidx], add=True)
    plsc.subcore_barrier()
    pltpu.sync_copy(degT.at[pl.ds(s * NPT, NPT)],
                    deg_out.at[c, pl.ds(s * NPT, NPT)])

    @pl.when(jnp.logical_and(c == 0, s == 0))
    def _():
        pltpu.sync_copy(ngT, ng_out)


_p1 = pl.kernel(
    _p1_body,
    out_type=(jax.ShapeDtypeStruct((NC, NPAD), jnp.float32),
              jax.ShapeDtypeStruct((NGP,), jnp.float32)),
    mesh=_mesh,
    scratch_types=(
        pltpu.VMEM_SHARED((NPAD,), jnp.float32),
        pltpu.VMEM_SHARED((NGP,), jnp.float32),
        pltpu.VMEM((NPT,), jnp.float32),
        pltpu.VMEM((K,), jnp.int32),
        pltpu.VMEM((NPT,), jnp.int32),
    ),
)


def _p2_body(s2_ref, d2_ref, v0_ref, v1_ref, v2_ref, z_ref,
             acc_out, accT0, accT1, accT2, sidx, didx, b0, b1, b2):
    c = lax.axis_index("c")
    s = lax.axis_index("s")
    wid = _wid()
    pltpu.sync_copy(z_ref.at[pl.ds(s * NPT, NPT)],
                    accT0.at[pl.ds(s * NPT, NPT)])
    pltpu.sync_copy(z_ref.at[pl.ds(s * NPT, NPT)],
                    accT1.at[pl.ds(s * NPT, NPT)])
    pltpu.sync_copy(z_ref.at[pl.ds(s * NPT, NPT)],
                    accT2.at[pl.ds(s * NPT, NPT)])
    plsc.subcore_barrier()

    def eblk(i, carry):
        base = wid * TPB + i * K
        pltpu.sync_copy(s2_ref.at[pl.ds(base, K)], sidx)
        pltpu.sync_copy(d2_ref.at[pl.ds(base, K)], didx)
        pltpu.sync_copy(v0_ref.at[sidx], b0)
        pltpu.sync_copy(b0, accT0.at[didx], add=True)
        pltpu.sync_copy(v1_ref.at[sidx], b1)
        pltpu.sync_copy(b1, accT1.at[didx], add=True)
        pltpu.sync_copy(v2_ref.at[sidx], b2)
        pltpu.sync_copy(b2, accT2.at[didx], add=True)
        return carry

    lax.fori_loop(0, NB, eblk, 0)
    plsc.subcore_barrier()
    pltpu.sync_copy(accT0.at[pl.ds(s * NPT, NPT)],
                    acc_out.at[c, 0, pl.ds(s * NPT, NPT)])
    pltpu.sync_copy(accT1.at[pl.ds(s * NPT, NPT)],
                    acc_out.at[c, 1, pl.ds(s * NPT, NPT)])
    pltpu.sync_copy(accT2.at[pl.ds(s * NPT, NPT)],
                    acc_out.at[c, 2, pl.ds(s * NPT, NPT)])


_p2 = pl.kernel(
    _p2_body,
    out_type=jax.ShapeDtypeStruct((NC, 3, NPAD), jnp.float32),
    mesh=_mesh,
    compiler_params=pltpu.CompilerParams(use_tc_tiling_on_sc=False),
    scratch_types=(
        pltpu.VMEM_SHARED((NPAD,), jnp.float32),
        pltpu.VMEM_SHARED((NPAD,), jnp.float32),
        pltpu.VMEM_SHARED((NPAD,), jnp.float32),
        pltpu.VMEM((K,), jnp.int32),
        pltpu.VMEM((K,), jnp.int32),
        pltpu.VMEM((K,), jnp.float32),
        pltpu.VMEM((K,), jnp.float32),
        pltpu.VMEM((K,), jnp.float32),
    ),
)


_UNROLL = 16


def _p3_body(s2_ref, d2_ref, u_ref, dinv_ref, batp_ref,
             p_out, sidx, didx, ubuf, wbuf, gbuf, P):
    c = lax.axis_index("c")
    s = lax.axis_index("s")
    wid = _wid()

    def zP(i, carry):
        P[i, :] = jnp.zeros((16,), jnp.float32)
        return carry

    lax.fori_loop(0, G + 1, zP, 0)

    def eblk(i, carry):
        base = wid * TPB + i * K
        pltpu.sync_copy(s2_ref.at[pl.ds(base, K)], sidx)
        pltpu.sync_copy(d2_ref.at[pl.ds(base, K)], didx)
        pltpu.sync_copy(u_ref.at[sidx], ubuf)
        pltpu.sync_copy(dinv_ref.at[didx], wbuf)
        pltpu.sync_copy(batp_ref.at[didx], gbuf)

        def edges(j, carry2):
            e0 = j * _UNROLL
            gv = gbuf[pl.ds(e0, 16)]
            wv = wbuf[pl.ds(e0, 16)]
            for t in range(_UNROLL):
                plsc.addupdate(P.at[gv[t]], ubuf[e0 + t, :] * wv[t])
            return carry2

        lax.fori_loop(0, K // _UNROLL, edges, 0)
        return carry

    lax.fori_loop(0, NB, eblk, 0)
    pltpu.sync_copy(P, p_out.at[c, s])


_p3 = pl.kernel(
    _p3_body,
    out_type=jax.ShapeDtypeStruct((NC, NS, G + 1, 16), jnp.float32),
    mesh=_mesh,
    compiler_params=pltpu.CompilerParams(use_tc_tiling_on_sc=False),
    scratch_types=(
        pltpu.VMEM((K,), jnp.int32),
        pltpu.VMEM((K,), jnp.int32),
        pltpu.VMEM((K, 16), jnp.float32),
        pltpu.VMEM((K,), jnp.float32),
        pltpu.VMEM((K,), jnp.int32),
        pltpu.VMEM((G + 1, 16), jnp.float32),
    ),
)


_BN = 2048


def _ka_body(deg_ref, x4_ref, dinv_ref, v0_ref, v1_ref, v2_ref):
    d = deg_ref[...]
    dv = jnp.where(d > 0, lax.rsqrt(d), 0.0)
    dinv_ref[...] = dv
    v = dv * x4_ref[...]
    v0_ref[...] = v[:, 0:1]
    v1_ref[...] = v[:, 1:2]
    v2_ref[...] = v[:, 2:3]


def _kb_body(a0_ref, a1_ref, a2_ref, dinv_ref, w1_ref, b1_ref, u_ref):
    dv = dinv_ref[...]
    h = ((dv * a0_ref[...]) * w1_ref[0:1, :]
         + (dv * a1_ref[...]) * w1_ref[1:2, :]
         + (dv * a2_ref[...]) * w1_ref[2:3, :]
         + b1_ref[...])
    h = jnp.maximum(h, 0.0)
    u_ref[...] = dv * h


def _kc_body(p_ref, ng_ref, w2_ref, b2_ref, wl_ref, bl_ref, out_ref):
    pp = jnp.sum(p_ref[...], axis=0)[:G, :]
    ng = ng_ref[...][:G, :]
    pooled = jnp.dot(pp, w2_ref[...], preferred_element_type=jnp.float32)
    pooled = pooled + ng * b2_ref[...]
    lg = jnp.dot(pooled, wl_ref[...], preferred_element_type=jnp.float32)
    lg = lg + bl_ref[...]
    m = jnp.max(lg, axis=1, keepdims=True)
    ls = jnp.log(jnp.sum(jnp.exp(lg - m), axis=1, keepdims=True))
    out_ref[...] = lg - m - ls


def kernel(x, edge_index, batch, W1, b1, W2, b2, Wlin, blin):
    src = edge_index[0]
    dst = edge_index[1]
    e = src.shape[0]
    loop = jnp.arange(N, dtype=jnp.int32)
    padlen = EPAD - e - N
    sinkpad = jnp.full((padlen,), N, dtype=jnp.int32)
    s2 = jnp.concatenate([src, loop, sinkpad])
    d2 = jnp.concatenate([dst, loop, sinkpad])
    batp = jnp.concatenate(
        [batch, jnp.full((NPAD - N,), G, dtype=jnp.int32)])
    x4 = jnp.pad(x, ((0, NPAD - N), (0, 1)))
    ones_h = jnp.ones((NPT,), jnp.float32)
    zeros_h = jnp.zeros((NPAD,), jnp.float32)

    deg2, ng = _p1(d2, batp, ones_h, zeros_h)
    degsum = (deg2[0] + deg2[1]).reshape(NPAD, 1)

    dinv, v0, v1, v2 = pl.pallas_call(
        _ka_body,
        grid=(NPAD // _BN,),
        in_specs=[pl.BlockSpec((_BN, 1), lambda i: (i, 0)),
                  pl.BlockSpec((_BN, 4), lambda i: (i, 0))],
        out_specs=[pl.BlockSpec((_BN, 1), lambda i: (i, 0))] * 4,
        out_shape=(jax.ShapeDtypeStruct((NPAD, 1), jnp.float32),) * 4,
    )(degsum, x4)

    acc2 = _p2(s2, d2, v0.reshape(NPAD), v1.reshape(NPAD),
               v2.reshape(NPAD), zeros_h)
    accsum = acc2[0] + acc2[1]

    u = pl.pallas_call(
        _kb_body,
        grid=(NPAD // _BN,),
        in_specs=[pl.BlockSpec((_BN, 1), lambda i: (i, 0))] * 4
        + [pl.BlockSpec((3, 16), lambda i: (0, 0)),
           pl.BlockSpec((1, 16), lambda i: (0, 0))],
        out_specs=pl.BlockSpec((_BN, 16), lambda i: (i, 0)),
        out_shape=jax.ShapeDtypeStruct((NPAD, 16), jnp.float32),
    )(accsum[0].reshape(NPAD, 1), accsum[1].reshape(NPAD, 1),
      accsum[2].reshape(NPAD, 1), dinv, W1, b1.reshape(1, 16))

    P = _p3(s2, d2, u, dinv.reshape(NPAD), batp)

    out = pl.pallas_call(
        _kc_body,
        out_shape=jax.ShapeDtypeStruct((G, 7), jnp.float32),
    )(P.reshape(NW, G + 1, 16), ng.reshape(NGP, 1),
      W2, b2.reshape(1, 16), Wlin, blin.reshape(1, 7))
    return out


# speedup vs baseline: 32.8189x; 1.1358x over previous
import jax
import jax.numpy as jnp
from jax import lax
from jax.experimental import pallas as pl
from jax.experimental.pallas import tpu as pltpu
from jax.experimental.pallas import tpu_sc as plsc

N = 100000
G = 256
NC, NS = 2, 16
NW = NC * NS
NPAD = 100352
NPT = NPAD // NS
K = 2048
NB = 100
TPB = K * NB
EPAD = TPB * NW
NGP = 384

_mesh = plsc.VectorSubcoreMesh(
    core_axis_name="c", subcore_axis_name="s", num_cores=NC, num_subcores=NS)


def _wid():
    return lax.axis_index("s") * NC + lax.axis_index("c")


def _p1_body(d2_ref, batp_ref, ones_ref, zeros_ref,
             deg_out, ng_out, degT, ngT, oneb, didx, bidx):
    c = lax.axis_index("c")
    s = lax.axis_index("s")
    wid = _wid()
    pltpu.sync_copy(ones_ref, oneb)
    pltpu.sync_copy(zeros_ref.at[pl.ds(s * NPT, NPT)],
                    degT.at[pl.ds(s * NPT, NPT)])

    @pl.when(jnp.logical_and(c == 0, s == 0))
    def _():
        pltpu.sync_copy(zeros_ref.at[pl.ds(0, NGP)], ngT)
    plsc.subcore_barrier()

    def eblk(i, carry):
        base = wid * TPB + i * K
        pltpu.sync_copy(d2_ref.at[pl.ds(base, K)], didx)
        pltpu.sync_copy(oneb.at[pl.ds(0, K)], degT.at[didx], add=True)
        return carry

    lax.fori_loop(0, NB, eblk, 0)

    @pl.when(c == 0)
    def _():
        pltpu.sync_copy(batp_ref.at[pl.ds(s * NPT, NPT)], bidx)
        pltpu.sync_copy(oneb, ngT.at[bidx], add=True)
    plsc.subcore_barrier()
    pltpu.sync_copy(degT.at[pl.ds(s * NPT, NPT)],
                    deg_out.at[c, pl.ds(s * NPT, NPT)])

    @pl.when(jnp.logical_and(c == 0, s == 0))
    def _():
        pltpu.sync_copy(ngT, ng_out)


_p1 = pl.kernel(
    _p1_body,
    out_type=(jax.ShapeDtypeStruct((NC, NPAD), jnp.float32),
              jax.ShapeDtypeStruct((NGP,), jnp.float32)),
    mesh=_mesh,
    scratch_types=(
        pltpu.VMEM_SHARED((NPAD,), jnp.float32),
        pltpu.VMEM_SHARED((NGP,), jnp.float32),
        pltpu.VMEM((NPT,), jnp.float32),
        pltpu.VMEM((K,), jnp.int32),
        pltpu.VMEM((NPT,), jnp.int32),
    ),
)


def _p2_body(s2_ref, d2_ref, v0_ref, v1_ref, v2_ref, z_ref,
             acc_out, accT0, accT1, accT2, sidx, didx, b0, b1, b2):
    c = lax.axis_index("c")
    s = lax.axis_index("s")
    wid = _wid()
    pltpu.sync_copy(z_ref.at[pl.ds(s * NPT, NPT)],
                    accT0.at[pl.ds(s * NPT, NPT)])
    pltpu.sync_copy(z_ref.at[pl.ds(s * NPT, NPT)],
                    accT1.at[pl.ds(s * NPT, NPT)])
    pltpu.sync_copy(z_ref.at[pl.ds(s * NPT, NPT)],
                    accT2.at[pl.ds(s * NPT, NPT)])
    plsc.subcore_barrier()

    def eblk(i, carry):
        base = wid * TPB + i * K
        pltpu.sync_copy(s2_ref.at[pl.ds(base, K)], sidx)
        pltpu.sync_copy(d2_ref.at[pl.ds(base, K)], didx)
        pltpu.sync_copy(v0_ref.at[sidx], b0)
        pltpu.sync_copy(b0, accT0.at[didx], add=True)
        pltpu.sync_copy(v1_ref.at[sidx], b1)
        pltpu.sync_copy(b1, accT1.at[didx], add=True)
        pltpu.sync_copy(v2_ref.at[sidx], b2)
        pltpu.sync_copy(b2, accT2.at[didx], add=True)
        return carry

    lax.fori_loop(0, NB, eblk, 0)
    plsc.subcore_barrier()
    pltpu.sync_copy(accT0.at[pl.ds(s * NPT, NPT)],
                    acc_out.at[c, 0, pl.ds(s * NPT, NPT)])
    pltpu.sync_copy(accT1.at[pl.ds(s * NPT, NPT)],
                    acc_out.at[c, 1, pl.ds(s * NPT, NPT)])
    pltpu.sync_copy(accT2.at[pl.ds(s * NPT, NPT)],
                    acc_out.at[c, 2, pl.ds(s * NPT, NPT)])


_p2 = pl.kernel(
    _p2_body,
    out_type=jax.ShapeDtypeStruct((NC, 3, NPAD), jnp.float32),
    mesh=_mesh,
    compiler_params=pltpu.CompilerParams(use_tc_tiling_on_sc=False),
    scratch_types=(
        pltpu.VMEM_SHARED((NPAD,), jnp.float32),
        pltpu.VMEM_SHARED((NPAD,), jnp.float32),
        pltpu.VMEM_SHARED((NPAD,), jnp.float32),
        pltpu.VMEM((K,), jnp.int32),
        pltpu.VMEM((K,), jnp.int32),
        pltpu.VMEM((K,), jnp.float32),
        pltpu.VMEM((K,), jnp.float32),
        pltpu.VMEM((K,), jnp.float32),
    ),
)


_UNROLL = 16
K3 = 2560
NB3 = TPB // K3


def _p3_body(s2_ref, d2_ref, u_ref, dinv_ref, batp_ref,
             p_out, sidx0, didx0, sidx1, didx1, ubuf0, ubuf1,
             wbuf0, wbuf1, gbuf0, gbuf1, P, sem0, sem1):
    c = lax.axis_index("c")
    s = lax.axis_index("s")
    wid = _wid()

    def zP(i, carry):
        P[i, :] = jnp.zeros((16,), jnp.float32)
        return carry

    lax.fori_loop(0, G + 1, zP, 0)

    def fetch(i, sidx, didx, ubuf, wbuf, gbuf, sem):
        base = wid * TPB + i * K3
        pltpu.sync_copy(s2_ref.at[pl.ds(base, K3)], sidx)
        pltpu.sync_copy(d2_ref.at[pl.ds(base, K3)], didx)
        du = pltpu.async_copy(u_ref.at[sidx], ubuf, sem)
        dw = pltpu.async_copy(dinv_ref.at[didx], wbuf, sem)
        dg = pltpu.async_copy(batp_ref.at[didx], gbuf, sem)
        return du, dw, dg

    def tec(ubuf, wbuf, gbuf):
        def edges(j, carry2):
            e0 = j * _UNROLL
            gv = gbuf[pl.ds(e0, 16)]
            wv = wbuf[pl.ds(e0, 16)]
            for t in range(_UNROLL):
                plsc.addupdate(P.at[gv[t]], ubuf[e0 + t, :] * wv[t])
            return carry2

        lax.fori_loop(0, K3 // _UNROLL, edges, 0)

    def pair(j, carry):
        d0 = fetch(2 * j, sidx0, didx0, ubuf0, wbuf0, gbuf0, sem0)
        d1 = fetch(2 * j + 1, sidx1, didx1, ubuf1, wbuf1, gbuf1, sem1)
        for d in d0:
            d.wait()
        tec(ubuf0, wbuf0, gbuf0)
        for d in d1:
            d.wait()
        tec(ubuf1, wbuf1, gbuf1)
        return carry

    lax.fori_loop(0, NB3 // 2, pair, 0)
    pltpu.sync_copy(P, p_out.at[c, s])


_p3 = pl.kernel(
    _p3_body,
    out_type=jax.ShapeDtypeStruct((NC, NS, G + 1, 16), jnp.float32),
    mesh=_mesh,
    compiler_params=pltpu.CompilerParams(use_tc_tiling_on_sc=False),
    scratch_types=(
        pltpu.VMEM((K3,), jnp.int32),
        pltpu.VMEM((K3,), jnp.int32),
        pltpu.VMEM((K3,), jnp.int32),
        pltpu.VMEM((K3,), jnp.int32),
        pltpu.VMEM((K3, 16), jnp.float32),
        pltpu.VMEM((K3, 16), jnp.float32),
        pltpu.VMEM((K3,), jnp.float32),
        pltpu.VMEM((K3,), jnp.float32),
        pltpu.VMEM((K3,), jnp.int32),
        pltpu.VMEM((K3,), jnp.int32),
        pltpu.VMEM((G + 1, 16), jnp.float32),
        pltpu.SemaphoreType.DMA,
        pltpu.SemaphoreType.DMA,
    ),
)


_BN = 2048


def _ka_body(deg_ref, x4_ref, dinv_ref, v0_ref, v1_ref, v2_ref):
    d = deg_ref[...]
    dv = jnp.where(d > 0, lax.rsqrt(d), 0.0)
    dinv_ref[...] = dv
    v = dv * x4_ref[...]
    v0_ref[...] = v[:, 0:1]
    v1_ref[...] = v[:, 1:2]
    v2_ref[...] = v[:, 2:3]


def _kb_body(a0_ref, a1_ref, a2_ref, dinv_ref, w1_ref, b1_ref, u_ref):
    dv = dinv_ref[...]
    h = ((dv * a0_ref[...]) * w1_ref[0:1, :]
         + (dv * a1_ref[...]) * w1_ref[1:2, :]
         + (dv * a2_ref[...]) * w1_ref[2:3, :]
         + b1_ref[...])
    h = jnp.maximum(h, 0.0)
    u_ref[...] = dv * h


def _kc_body(p_ref, ng_ref, w2_ref, b2_ref, wl_ref, bl_ref, out_ref):
    pp = jnp.sum(p_ref[...], axis=0)[:G, :]
    ng = ng_ref[...][:G, :]
    pooled = jnp.dot(pp, w2_ref[...], preferred_element_type=jnp.float32)
    pooled = pooled + ng * b2_ref[...]
    lg = jnp.dot(pooled, wl_ref[...], preferred_element_type=jnp.float32)
    lg = lg + bl_ref[...]
    m = jnp.max(lg, axis=1, keepdims=True)
    ls = jnp.log(jnp.sum(jnp.exp(lg - m), axis=1, keepdims=True))
    out_ref[...] = lg - m - ls


def kernel(x, edge_index, batch, W1, b1, W2, b2, Wlin, blin):
    src = edge_index[0]
    dst = edge_index[1]
    e = src.shape[0]
    loop = jnp.arange(N, dtype=jnp.int32)
    padlen = EPAD - e - N
    sinkpad = jnp.full((padlen,), N, dtype=jnp.int32)
    s2 = jnp.concatenate([src, loop, sinkpad])
    d2 = jnp.concatenate([dst, loop, sinkpad])
    batp = jnp.concatenate(
        [batch, jnp.full((NPAD - N,), G, dtype=jnp.int32)])
    x4 = jnp.pad(x, ((0, NPAD - N), (0, 1)))
    ones_h = jnp.ones((NPT,), jnp.float32)
    zeros_h = jnp.zeros((NPAD,), jnp.float32)

    deg2, ng = _p1(d2, batp, ones_h, zeros_h)
    degsum = (deg2[0] + deg2[1]).reshape(NPAD, 1)

    dinv, v0, v1, v2 = pl.pallas_call(
        _ka_body,
        grid=(NPAD // _BN,),
        in_specs=[pl.BlockSpec((_BN, 1), lambda i: (i, 0)),
                  pl.BlockSpec((_BN, 4), lambda i: (i, 0))],
        out_specs=[pl.BlockSpec((_BN, 1), lambda i: (i, 0))] * 4,
        out_shape=(jax.ShapeDtypeStruct((NPAD, 1), jnp.float32),) * 4,
    )(degsum, x4)

    acc2 = _p2(s2, d2, v0.reshape(NPAD), v1.reshape(NPAD),
               v2.reshape(NPAD), zeros_h)
    accsum = acc2[0] + acc2[1]

    u = pl.pallas_call(
        _kb_body,
        grid=(NPAD // _BN,),
        in_specs=[pl.BlockSpec((_BN, 1), lambda i: (i, 0))] * 4
        + [pl.BlockSpec((3, 16), lambda i: (0, 0)),
           pl.BlockSpec((1, 16), lambda i: (0, 0))],
        out_specs=pl.BlockSpec((_BN, 16), lambda i: (i, 0)),
        out_shape=jax.ShapeDtypeStruct((NPAD, 16), jnp.float32),
    )(accsum[0].reshape(NPAD, 1), accsum[1].reshape(NPAD, 1),
      accsum[2].reshape(NPAD, 1), dinv, W1, b1.reshape(1, 16))

    P = _p3(s2, d2, u, dinv.reshape(NPAD), batp)

    out = pl.pallas_call(
        _kc_body,
        out_shape=jax.ShapeDtypeStruct((G, 7), jnp.float32),
    )(P.reshape(NW, G + 1, 16), ng.reshape(NGP, 1),
      W2, b2.reshape(1, 16), Wlin, blin.reshape(1, 7))
    return out


# speedup vs baseline: 35.2332x; 1.0736x over previous
import jax
import jax.numpy as jnp
from jax import lax
from jax.experimental import pallas as pl
from jax.experimental.pallas import tpu as pltpu
from jax.experimental.pallas import tpu_sc as plsc

N = 100000
G = 256
NC, NS = 2, 16
NW = NC * NS
NPAD = 100352
NPT = NPAD // NS
TPB = 204800
EPAD = TPB * NW
NGP = 384

_mesh = plsc.VectorSubcoreMesh(
    core_axis_name="c", subcore_axis_name="s", num_cores=NC, num_subcores=NS)


def _wid():
    return lax.axis_index("s") * NC + lax.axis_index("c")


K1 = 12800
NB1 = TPB // K1


def _p1_body(d2_ref, batp_ref, ones_ref, zeros_ref,
             deg_out, ng_out, degT, ngT, oneb, didx0, didx1, bidx,
             sem0, sem1):
    c = lax.axis_index("c")
    s = lax.axis_index("s")
    wid = _wid()
    pltpu.sync_copy(ones_ref, oneb)
    pltpu.sync_copy(zeros_ref.at[pl.ds(s * NPT, NPT)],
                    degT.at[pl.ds(s * NPT, NPT)])

    @pl.when(jnp.logical_and(c == 0, s == 0))
    def _():
        pltpu.sync_copy(zeros_ref.at[pl.ds(0, NGP)], ngT)
    plsc.subcore_barrier()

    def eblk(j, carry):
        base = wid * TPB + 2 * j * K1
        pltpu.sync_copy(d2_ref.at[pl.ds(base, K1)], didx0)
        d0 = pltpu.async_copy(oneb.at[pl.ds(0, K1)], degT.at[didx0], sem0,
                              add=True)
        pltpu.sync_copy(d2_ref.at[pl.ds(base + K1, K1)], didx1)
        d1 = pltpu.async_copy(oneb.at[pl.ds(0, K1)], degT.at[didx1], sem1,
                              add=True)
        d0.wait()
        d1.wait()
        return carry

    lax.fori_loop(0, NB1 // 2, eblk, 0)

    @pl.when(c == 0)
    def _():
        pltpu.sync_copy(batp_ref.at[pl.ds(s * NPT, NPT)], bidx)
        pltpu.sync_copy(oneb.at[pl.ds(0, NPT)], ngT.at[bidx], add=True)
    plsc.subcore_barrier()
    pltpu.sync_copy(degT.at[pl.ds(s * NPT, NPT)],
                    deg_out.at[c, pl.ds(s * NPT, NPT)])

    @pl.when(jnp.logical_and(c == 0, s == 0))
    def _():
        pltpu.sync_copy(ngT, ng_out)


_p1 = pl.kernel(
    _p1_body,
    out_type=(jax.ShapeDtypeStruct((NC, NPAD), jnp.float32),
              jax.ShapeDtypeStruct((NGP,), jnp.float32)),
    mesh=_mesh,
    scratch_types=(
        pltpu.VMEM_SHARED((NPAD,), jnp.float32),
        pltpu.VMEM_SHARED((NGP,), jnp.float32),
        pltpu.VMEM((K1,), jnp.float32),
        pltpu.VMEM((K1,), jnp.int32),
        pltpu.VMEM((K1,), jnp.int32),
        pltpu.VMEM((NPT,), jnp.int32),
        pltpu.SemaphoreType.DMA,
        pltpu.SemaphoreType.DMA,
    ),
)


K2 = 6400
NB2 = TPB // K2
def _p2_body(s2_ref, d2_ref, v0_ref, v1_ref, v2_ref, z_ref,
             acc_out, accT0, accT1, accT2,
             sidx0, didx0, sidx1, didx1,
             b00, b01, b02, b10, b11, b12, semg0, semg1, sems0, sems1):
    c = lax.axis_index("c")
    s = lax.axis_index("s")
    wid = _wid()
    pltpu.sync_copy(z_ref.at[pl.ds(s * NPT, NPT)],
                    accT0.at[pl.ds(s * NPT, NPT)])
    pltpu.sync_copy(z_ref.at[pl.ds(s * NPT, NPT)],
                    accT1.at[pl.ds(s * NPT, NPT)])
    pltpu.sync_copy(z_ref.at[pl.ds(s * NPT, NPT)],
                    accT2.at[pl.ds(s * NPT, NPT)])
    plsc.subcore_barrier()

    def gather(i, sidx, didx, b0, b1, b2, sem):
        base = wid * TPB + i * K2
        pltpu.sync_copy(s2_ref.at[pl.ds(base, K2)], sidx)
        pltpu.sync_copy(d2_ref.at[pl.ds(base, K2)], didx)
        return (pltpu.async_copy(v0_ref.at[sidx], b0, sem),
                pltpu.async_copy(v1_ref.at[sidx], b1, sem),
                pltpu.async_copy(v2_ref.at[sidx], b2, sem))

    def scatter(didx, b0, b1, b2, sem):
        return (pltpu.async_copy(b0, accT0.at[didx], sem, add=True),
                pltpu.async_copy(b1, accT1.at[didx], sem, add=True),
                pltpu.async_copy(b2, accT2.at[didx], sem, add=True))

    def eblk(j, carry):
        g0 = gather(2 * j, sidx0, didx0, b00, b01, b02, semg0)
        g1 = gather(2 * j + 1, sidx1, didx1, b10, b11, b12, semg1)
        for d in g0:
            d.wait()
        s0 = scatter(didx0, b00, b01, b02, sems0)
        for d in g1:
            d.wait()
        s1 = scatter(didx1, b10, b11, b12, sems1)
        for d in s0:
            d.wait()
        for d in s1:
            d.wait()
        return carry

    lax.fori_loop(0, NB2 // 2, eblk, 0)
    plsc.subcore_barrier()
    pltpu.sync_copy(accT0.at[pl.ds(s * NPT, NPT)],
                    acc_out.at[c, 0, pl.ds(s * NPT, NPT)])
    pltpu.sync_copy(accT1.at[pl.ds(s * NPT, NPT)],
                    acc_out.at[c, 1, pl.ds(s * NPT, NPT)])
    pltpu.sync_copy(accT2.at[pl.ds(s * NPT, NPT)],
                    acc_out.at[c, 2, pl.ds(s * NPT, NPT)])


_p2 = pl.kernel(
    _p2_body,
    out_type=jax.ShapeDtypeStruct((NC, 3, NPAD), jnp.float32),
    mesh=_mesh,
    compiler_params=pltpu.CompilerParams(use_tc_tiling_on_sc=False),
    scratch_types=(
        pltpu.VMEM_SHARED((NPAD,), jnp.float32),
        pltpu.VMEM_SHARED((NPAD,), jnp.float32),
        pltpu.VMEM_SHARED((NPAD,), jnp.float32),
        pltpu.VMEM((K2,), jnp.int32),
        pltpu.VMEM((K2,), jnp.int32),
        pltpu.VMEM((K2,), jnp.int32),
        pltpu.VMEM((K2,), jnp.int32),
        pltpu.VMEM((K2,), jnp.float32),
        pltpu.VMEM((K2,), jnp.float32),
        pltpu.VMEM((K2,), jnp.float32),
        pltpu.VMEM((K2,), jnp.float32),
        pltpu.VMEM((K2,), jnp.float32),
        pltpu.VMEM((K2,), jnp.float32),
        pltpu.SemaphoreType.DMA,
        pltpu.SemaphoreType.DMA,
        pltpu.SemaphoreType.DMA,
        pltpu.SemaphoreType.DMA,
    ),
)


_UNROLL = 16
K3 = 2560
NB3 = TPB // K3


def _p3_body(s2_ref, d2_ref, u_ref, dinv_ref, batp_ref,
             p_out, sidx0, didx0, sidx1, didx1, ubuf0, ubuf1,
             wbuf0, wbuf1, gbuf0, gbuf1, P, sem0, sem1):
    c = lax.axis_index("c")
    s = lax.axis_index("s")
    wid = _wid()

    def zP(i, carry):
        P[i, :] = jnp.zeros((16,), jnp.float32)
        return carry

    lax.fori_loop(0, G + 1, zP, 0)

    def fetch(i, sidx, didx, ubuf, wbuf, gbuf, sem):
        base = wid * TPB + i * K3
        pltpu.sync_copy(s2_ref.at[pl.ds(base, K3)], sidx)
        pltpu.sync_copy(d2_ref.at[pl.ds(base, K3)], didx)
        du = pltpu.async_copy(u_ref.at[sidx], ubuf, sem)
        dw = pltpu.async_copy(dinv_ref.at[didx], wbuf, sem)
        dg = pltpu.async_copy(batp_ref.at[didx], gbuf, sem)
        return du, dw, dg

    def tec(ubuf, wbuf, gbuf):
        def edges(j, carry2):
            e0 = j * _UNROLL
            gv = gbuf[pl.ds(e0, 16)]
            wv = wbuf[pl.ds(e0, 16)]
            for t in range(_UNROLL):
                plsc.addupdate(P.at[gv[t]], ubuf[e0 + t, :] * wv[t])
            return carry2

        lax.fori_loop(0, K3 // _UNROLL, edges, 0)

    def pair(j, carry):
        d0 = fetch(2 * j, sidx0, didx0, ubuf0, wbuf0, gbuf0, sem0)
        d1 = fetch(2 * j + 1, sidx1, didx1, ubuf1, wbuf1, gbuf1, sem1)
        for d in d0:
            d.wait()
        tec(ubuf0, wbuf0, gbuf0)
        for d in d1:
            d.wait()
        tec(ubuf1, wbuf1, gbuf1)
        return carry

    lax.fori_loop(0, NB3 // 2, pair, 0)
    pltpu.sync_copy(P, p_out.at[c, s])


_p3 = pl.kernel(
    _p3_body,
    out_type=jax.ShapeDtypeStruct((NC, NS, G + 1, 16), jnp.float32),
    mesh=_mesh,
    compiler_params=pltpu.CompilerParams(use_tc_tiling_on_sc=False),
    scratch_types=(
        pltpu.VMEM((K3,), jnp.int32),
        pltpu.VMEM((K3,), jnp.int32),
        pltpu.VMEM((K3,), jnp.int32),
        pltpu.VMEM((K3,), jnp.int32),
        pltpu.VMEM((K3, 16), jnp.float32),
        pltpu.VMEM((K3, 16), jnp.float32),
        pltpu.VMEM((K3,), jnp.float32),
        pltpu.VMEM((K3,), jnp.float32),
        pltpu.VMEM((K3,), jnp.int32),
        pltpu.VMEM((K3,), jnp.int32),
        pltpu.VMEM((G + 1, 16), jnp.float32),
        pltpu.SemaphoreType.DMA,
        pltpu.SemaphoreType.DMA,
    ),
)


_BN = 2048


def _ka_body(deg_ref, x4_ref, dinv_ref, v0_ref, v1_ref, v2_ref):
    d = deg_ref[...]
    dv = jnp.where(d > 0, lax.rsqrt(d), 0.0)
    dinv_ref[...] = dv
    v = dv * x4_ref[...]
    v0_ref[...] = v[:, 0:1]
    v1_ref[...] = v[:, 1:2]
    v2_ref[...] = v[:, 2:3]


def _kb_body(a0_ref, a1_ref, a2_ref, dinv_ref, w1_ref, b1_ref, u_ref):
    dv = dinv_ref[...]
    h = ((dv * a0_ref[...]) * w1_ref[0:1, :]
         + (dv * a1_ref[...]) * w1_ref[1:2, :]
         + (dv * a2_ref[...]) * w1_ref[2:3, :]
         + b1_ref[...])
    h = jnp.maximum(h, 0.0)
    u_ref[...] = dv * h


def _kc_body(p_ref, ng_ref, w2_ref, b2_ref, wl_ref, bl_ref, out_ref):
    pp = jnp.sum(p_ref[...], axis=0)[:G, :]
    ng = ng_ref[...][:G, :]
    pooled = jnp.dot(pp, w2_ref[...], preferred_element_type=jnp.float32)
    pooled = pooled + ng * b2_ref[...]
    lg = jnp.dot(pooled, wl_ref[...], preferred_element_type=jnp.float32)
    lg = lg + bl_ref[...]
    m = jnp.max(lg, axis=1, keepdims=True)
    ls = jnp.log(jnp.sum(jnp.exp(lg - m), axis=1, keepdims=True))
    out_ref[...] = lg - m - ls


def kernel(x, edge_index, batch, W1, b1, W2, b2, Wlin, blin):
    src = edge_index[0]
    dst = edge_index[1]
    e = src.shape[0]
    loop = jnp.arange(N, dtype=jnp.int32)
    padlen = EPAD - e - N
    sinkpad = jnp.full((padlen,), N, dtype=jnp.int32)
    s2 = jnp.concatenate([src, loop, sinkpad])
    d2 = jnp.concatenate([dst, loop, sinkpad])
    batp = jnp.concatenate(
        [batch, jnp.full((NPAD - N,), G, dtype=jnp.int32)])
    x4 = jnp.pad(x, ((0, NPAD - N), (0, 1)))
    ones_h = jnp.ones((K1,), jnp.float32)
    zeros_h = jnp.zeros((NPAD,), jnp.float32)

    deg2, ng = _p1(d2, batp, ones_h, zeros_h)
    degsum = (deg2[0] + deg2[1]).reshape(NPAD, 1)

    dinv, v0, v1, v2 = pl.pallas_call(
        _ka_body,
        grid=(NPAD // _BN,),
        in_specs=[pl.BlockSpec((_BN, 1), lambda i: (i, 0)),
                  pl.BlockSpec((_BN, 4), lambda i: (i, 0))],
        out_specs=[pl.BlockSpec((_BN, 1), lambda i: (i, 0))] * 4,
        out_shape=(jax.ShapeDtypeStruct((NPAD, 1), jnp.float32),) * 4,
    )(degsum, x4)

    acc2 = _p2(s2, d2, v0.reshape(NPAD), v1.reshape(NPAD),
               v2.reshape(NPAD), zeros_h)
    accsum = acc2[0] + acc2[1]

    u = pl.pallas_call(
        _kb_body,
        grid=(NPAD // _BN,),
        in_specs=[pl.BlockSpec((_BN, 1), lambda i: (i, 0))] * 4
        + [pl.BlockSpec((3, 16), lambda i: (0, 0)),
           pl.BlockSpec((1, 16), lambda i: (0, 0))],
        out_specs=pl.BlockSpec((_BN, 16), lambda i: (i, 0)),
        out_shape=jax.ShapeDtypeStruct((NPAD, 16), jnp.float32),
    )(accsum[0].reshape(NPAD, 1), accsum[1].reshape(NPAD, 1),
      accsum[2].reshape(NPAD, 1), dinv, W1, b1.reshape(1, 16))

    P = _p3(s2, d2, u, dinv.reshape(NPAD), batp)

    out = pl.pallas_call(
        _kc_body,
        out_shape=jax.ShapeDtypeStruct((G, 7), jnp.float32),
    )(P.reshape(NW, G + 1, 16), ng.reshape(NGP, 1),
      W2, b2.reshape(1, 16), Wlin, blin.reshape(1, 7))
    return out


# speedup vs baseline: 37.1787x; 1.0552x over previous
import jax
import jax.numpy as jnp
from jax import lax
from jax.experimental import pallas as pl
from jax.experimental.pallas import tpu as pltpu
from jax.experimental.pallas import tpu_sc as plsc

N = 100000
G = 256
NC, NS = 2, 16
NW = NC * NS
NPAD = 100352
NPT = NPAD // NS
TPB = 204800
EPAD = TPB * NW
T0 = 9 * 25600
T1 = 7 * 25600
OFF1 = NS * T0


def _ebase(c, s):
    return jnp.where(c == 0, s * T0, OFF1 + s * T1)
NGP = 384

_mesh = plsc.VectorSubcoreMesh(
    core_axis_name="c", subcore_axis_name="s", num_cores=NC, num_subcores=NS)


def _wid():
    return lax.axis_index("s") * NC + lax.axis_index("c")


K1 = 12800
NB1 = TPB // K1


def _p1_body(d2_ref, batp_ref, ones_ref, zeros_ref,
             deg_out, ng_out, degT, ngT, oneb, didx0, didx1, bidx,
             sem0, sem1):
    c = lax.axis_index("c")
    s = lax.axis_index("s")
    wid = _wid()
    pltpu.sync_copy(ones_ref, oneb)
    pltpu.sync_copy(zeros_ref.at[pl.ds(s * NPT, NPT)],
                    degT.at[pl.ds(s * NPT, NPT)])

    @pl.when(jnp.logical_and(c == 0, s == 0))
    def _():
        pltpu.sync_copy(zeros_ref.at[pl.ds(0, NGP)], ngT)
    plsc.subcore_barrier()

    eb = _ebase(c, s)

    def eblk(j, carry):
        base = eb + 2 * j * K1
        pltpu.sync_copy(d2_ref.at[pl.ds(base, K1)], didx0)
        d0 = pltpu.async_copy(oneb.at[pl.ds(0, K1)], degT.at[didx0], sem0,
                              add=True)
        pltpu.sync_copy(d2_ref.at[pl.ds(base + K1, K1)], didx1)
        d1 = pltpu.async_copy(oneb.at[pl.ds(0, K1)], degT.at[didx1], sem1,
                              add=True)
        d0.wait()
        d1.wait()
        return carry

    @pl.when(c == 0)
    def _():
        lax.fori_loop(0, T0 // K1 // 2, eblk, 0)

    @pl.when(c != 0)
    def _():
        lax.fori_loop(0, T1 // K1 // 2, eblk, 0)

    @pl.when(c == 0)
    def _():
        pltpu.sync_copy(batp_ref.at[pl.ds(s * NPT, NPT)], bidx)
        pltpu.sync_copy(oneb.at[pl.ds(0, NPT)], ngT.at[bidx], add=True)
    plsc.subcore_barrier()
    pltpu.sync_copy(degT.at[pl.ds(s * NPT, NPT)],
                    deg_out.at[c, pl.ds(s * NPT, NPT)])

    @pl.when(jnp.logical_and(c == 0, s == 0))
    def _():
        pltpu.sync_copy(ngT, ng_out)


_p1 = pl.kernel(
    _p1_body,
    out_type=(jax.ShapeDtypeStruct((NC, NPAD), jnp.float32),
              jax.ShapeDtypeStruct((NGP,), jnp.float32)),
    mesh=_mesh,
    scratch_types=(
        pltpu.VMEM_SHARED((NPAD,), jnp.float32),
        pltpu.VMEM_SHARED((NGP,), jnp.float32),
        pltpu.VMEM((K1,), jnp.float32),
        pltpu.VMEM((K1,), jnp.int32),
        pltpu.VMEM((K1,), jnp.int32),
        pltpu.VMEM((NPT,), jnp.int32),
        pltpu.SemaphoreType.DMA,
        pltpu.SemaphoreType.DMA,
    ),
)


K2 = 6400
NB2 = TPB // K2
def _p2_body(s2_ref, d2_ref, v0_ref, v1_ref, v2_ref, z_ref,
             acc_out, accT0, accT1, accT2,
             sidx0, didx0, sidx1, didx1,
             b00, b01, b02, b10, b11, b12, semg0, semg1, sems0, sems1):
    c = lax.axis_index("c")
    s = lax.axis_index("s")
    wid = _wid()
    pltpu.sync_copy(z_ref.at[pl.ds(s * NPT, NPT)],
                    accT0.at[pl.ds(s * NPT, NPT)])
    pltpu.sync_copy(z_ref.at[pl.ds(s * NPT, NPT)],
                    accT1.at[pl.ds(s * NPT, NPT)])
    pltpu.sync_copy(z_ref.at[pl.ds(s * NPT, NPT)],
                    accT2.at[pl.ds(s * NPT, NPT)])
    plsc.subcore_barrier()

    eb = _ebase(c, s)

    def gather(i, sidx, didx, b0, b1, b2, sem):
        base = eb + i * K2
        pltpu.sync_copy(s2_ref.at[pl.ds(base, K2)], sidx)
        pltpu.sync_copy(d2_ref.at[pl.ds(base, K2)], didx)
        return (pltpu.async_copy(v0_ref.at[sidx], b0, sem),
                pltpu.async_copy(v1_ref.at[sidx], b1, sem),
                pltpu.async_copy(v2_ref.at[sidx], b2, sem))

    def scatter(didx, b0, b1, b2, sem):
        return (pltpu.async_copy(b0, accT0.at[didx], sem, add=True),
                pltpu.async_copy(b1, accT1.at[didx], sem, add=True),
                pltpu.async_copy(b2, accT2.at[didx], sem, add=True))

    def eblk(j, carry):
        g0 = gather(2 * j, sidx0, didx0, b00, b01, b02, semg0)
        g1 = gather(2 * j + 1, sidx1, didx1, b10, b11, b12, semg1)
        for d in g0:
            d.wait()
        s0 = scatter(didx0, b00, b01, b02, sems0)
        for d in g1:
            d.wait()
        s1 = scatter(didx1, b10, b11, b12, sems1)
        for d in s0:
            d.wait()
        for d in s1:
            d.wait()
        return carry

    @pl.when(c == 0)
    def _():
        lax.fori_loop(0, T0 // K2 // 2, eblk, 0)

    @pl.when(c != 0)
    def _():
        lax.fori_loop(0, T1 // K2 // 2, eblk, 0)
    plsc.subcore_barrier()
    pltpu.sync_copy(accT0.at[pl.ds(s * NPT, NPT)],
                    acc_out.at[c, 0, pl.ds(s * NPT, NPT)])
    pltpu.sync_copy(accT1.at[pl.ds(s * NPT, NPT)],
                    acc_out.at[c, 1, pl.ds(s * NPT, NPT)])
    pltpu.sync_copy(accT2.at[pl.ds(s * NPT, NPT)],
                    acc_out.at[c, 2, pl.ds(s * NPT, NPT)])


_p2 = pl.kernel(
    _p2_body,
    out_type=jax.ShapeDtypeStruct((NC, 3, NPAD), jnp.float32),
    mesh=_mesh,
    compiler_params=pltpu.CompilerParams(use_tc_tiling_on_sc=False),
    scratch_types=(
        pltpu.VMEM_SHARED((NPAD,), jnp.float32),
        pltpu.VMEM_SHARED((NPAD,), jnp.float32),
        pltpu.VMEM_SHARED((NPAD,), jnp.float32),
        pltpu.VMEM((K2,), jnp.int32),
        pltpu.VMEM((K2,), jnp.int32),
        pltpu.VMEM((K2,), jnp.int32),
        pltpu.VMEM((K2,), jnp.int32),
        pltpu.VMEM((K2,), jnp.float32),
        pltpu.VMEM((K2,), jnp.float32),
        pltpu.VMEM((K2,), jnp.float32),
        pltpu.VMEM((K2,), jnp.float32),
        pltpu.VMEM((K2,), jnp.float32),
        pltpu.VMEM((K2,), jnp.float32),
        pltpu.SemaphoreType.DMA,
        pltpu.SemaphoreType.DMA,
        pltpu.SemaphoreType.DMA,
        pltpu.SemaphoreType.DMA,
    ),
)


_UNROLL = 16
K3 = 2560
NB3 = TPB // K3


def _p3_body(s2_ref, d2_ref, u_ref, dinv_ref, batp_ref,
             p_out, sidx0, didx0, sidx1, didx1, ubuf0, ubuf1,
             wbuf0, wbuf1, gbuf0, gbuf1, P, sem0, sem1):
    c = lax.axis_index("c")
    s = lax.axis_index("s")
    wid = _wid()

    def zP(i, carry):
        P[i, :] = jnp.zeros((16,), jnp.float32)
        return carry

    lax.fori_loop(0, G + 1, zP, 0)

    eb = _ebase(c, s)

    def fetch(i, sidx, didx, ubuf, wbuf, gbuf, sem):
        base = eb + i * K3
        pltpu.sync_copy(s2_ref.at[pl.ds(base, K3)], sidx)
        pltpu.sync_copy(d2_ref.at[pl.ds(base, K3)], didx)
        du = pltpu.async_copy(u_ref.at[sidx], ubuf, sem)
        dw = pltpu.async_copy(dinv_ref.at[didx], wbuf, sem)
        dg = pltpu.async_copy(batp_ref.at[didx], gbuf, sem)
        return du, dw, dg

    def tec(ubuf, wbuf, gbuf):
        def edges(j, carry2):
            e0 = j * _UNROLL
            gv = gbuf[pl.ds(e0, 16)]
            wv = wbuf[pl.ds(e0, 16)]
            for t in range(_UNROLL):
                plsc.addupdate(P.at[gv[t]], ubuf[e0 + t, :] * wv[t])
            return carry2

        lax.fori_loop(0, K3 // _UNROLL, edges, 0)

    def pair(j, carry):
        d0 = fetch(2 * j, sidx0, didx0, ubuf0, wbuf0, gbuf0, sem0)
        d1 = fetch(2 * j + 1, sidx1, didx1, ubuf1, wbuf1, gbuf1, sem1)
        for d in d0:
            d.wait()
        tec(ubuf0, wbuf0, gbuf0)
        for d in d1:
            d.wait()
        tec(ubuf1, wbuf1, gbuf1)
        return carry

    @pl.when(c == 0)
    def _():
        lax.fori_loop(0, T0 // K3 // 2, pair, 0)

    @pl.when(c != 0)
    def _():
        lax.fori_loop(0, T1 // K3 // 2, pair, 0)
    pltpu.sync_copy(P, p_out.at[c, s])


_p3 = pl.kernel(
    _p3_body,
    out_type=jax.ShapeDtypeStruct((NC, NS, G + 1, 16), jnp.float32),
    mesh=_mesh,
    compiler_params=pltpu.CompilerParams(use_tc_tiling_on_sc=False),
    scratch_types=(
        pltpu.VMEM((K3,), jnp.int32),
        pltpu.VMEM((K3,), jnp.int32),
        pltpu.VMEM((K3,), jnp.int32),
        pltpu.VMEM((K3,), jnp.int32),
        pltpu.VMEM((K3, 16), jnp.float32),
        pltpu.VMEM((K3, 16), jnp.float32),
        pltpu.VMEM((K3,), jnp.float32),
        pltpu.VMEM((K3,), jnp.float32),
        pltpu.VMEM((K3,), jnp.int32),
        pltpu.VMEM((K3,), jnp.int32),
        pltpu.VMEM((G + 1, 16), jnp.float32),
        pltpu.SemaphoreType.DMA,
        pltpu.SemaphoreType.DMA,
    ),
)


_BN = 2048


def _ka_body(deg_ref, x4_ref, dinv_ref, v0_ref, v1_ref, v2_ref):
    d = deg_ref[...]
    dv = jnp.where(d > 0, lax.rsqrt(d), 0.0)
    dinv_ref[...] = dv
    v = dv * x4_ref[...]
    v0_ref[...] = v[:, 0:1]
    v1_ref[...] = v[:, 1:2]
    v2_ref[...] = v[:, 2:3]


def _kb_body(a0_ref, a1_ref, a2_ref, dinv_ref, w1_ref, b1_ref, u_ref):
    dv = dinv_ref[...]
    h = ((dv * a0_ref[...]) * w1_ref[0:1, :]
         + (dv * a1_ref[...]) * w1_ref[1:2, :]
         + (dv * a2_ref[...]) * w1_ref[2:3, :]
         + b1_ref[...])
    h = jnp.maximum(h, 0.0)
    u_ref[...] = dv * h


def _kc_body(p_ref, ng_ref, w2_ref, b2_ref, wl_ref, bl_ref, out_ref):
    pp = jnp.sum(p_ref[...], axis=0)[:G, :]
    ng = ng_ref[...][:G, :]
    pooled = jnp.dot(pp, w2_ref[...], preferred_element_type=jnp.float32)
    pooled = pooled + ng * b2_ref[...]
    lg = jnp.dot(pooled, wl_ref[...], preferred_element_type=jnp.float32)
    lg = lg + bl_ref[...]
    m = jnp.max(lg, axis=1, keepdims=True)
    ls = jnp.log(jnp.sum(jnp.exp(lg - m), axis=1, keepdims=True))
    out_ref[...] = lg - m - ls


def kernel(x, edge_index, batch, W1, b1, W2, b2, Wlin, blin):
    src = edge_index[0]
    dst = edge_index[1]
    e = src.shape[0]
    loop = jnp.arange(N, dtype=jnp.int32)
    padlen = EPAD - e - N
    sinkpad = jnp.full((padlen,), N, dtype=jnp.int32)
    s2 = jnp.concatenate([src, loop, sinkpad])
    d2 = jnp.concatenate([dst, loop, sinkpad])
    batp = jnp.concatenate(
        [batch, jnp.full((NPAD - N,), G, dtype=jnp.int32)])
    x4 = jnp.pad(x, ((0, NPAD - N), (0, 1)))
    ones_h = jnp.ones((K1,), jnp.float32)
    zeros_h = jnp.zeros((NPAD,), jnp.float32)

    deg2, ng = _p1(d2, batp, ones_h, zeros_h)
    degsum = (deg2[0] + deg2[1]).reshape(NPAD, 1)

    dinv, v0, v1, v2 = pl.pallas_call(
        _ka_body,
        grid=(NPAD // _BN,),
        in_specs=[pl.BlockSpec((_BN, 1), lambda i: (i, 0)),
                  pl.BlockSpec((_BN, 4), lambda i: (i, 0))],
        out_specs=[pl.BlockSpec((_BN, 1), lambda i: (i, 0))] * 4,
        out_shape=(jax.ShapeDtypeStruct((NPAD, 1), jnp.float32),) * 4,
    )(degsum, x4)

    acc2 = _p2(s2, d2, v0.reshape(NPAD), v1.reshape(NPAD),
               v2.reshape(NPAD), zeros_h)
    accsum = acc2[0] + acc2[1]

    u = pl.pallas_call(
        _kb_body,
        grid=(NPAD // _BN,),
        in_specs=[pl.BlockSpec((_BN, 1), lambda i: (i, 0))] * 4
        + [pl.BlockSpec((3, 16), lambda i: (0, 0)),
           pl.BlockSpec((1, 16), lambda i: (0, 0))],
        out_specs=pl.BlockSpec((_BN, 16), lambda i: (i, 0)),
        out_shape=jax.ShapeDtypeStruct((NPAD, 16), jnp.float32),
    )(accsum[0].reshape(NPAD, 1), accsum[1].reshape(NPAD, 1),
      accsum[2].reshape(NPAD, 1), dinv, W1, b1.reshape(1, 16))

    P = _p3(s2, d2, u, dinv.reshape(NPAD), batp)

    out = pl.pallas_call(
        _kc_body,
        out_shape=jax.ShapeDtypeStruct((G, 7), jnp.float32),
    )(P.reshape(NW, G + 1, 16), ng.reshape(NGP, 1),
      W2, b2.reshape(1, 16), Wlin, blin.reshape(1, 7))
    return out


# speedup vs baseline: 37.2118x; 1.0009x over previous
import jax
import jax.numpy as jnp
from jax import lax
from jax.experimental import pallas as pl
from jax.experimental.pallas import tpu as pltpu
from jax.experimental.pallas import tpu_sc as plsc

N = 100000
G = 256
NC, NS = 2, 16
NW = NC * NS
NPAD = 100352
NPT = NPAD // NS
TPB = 204800
EPAD = TPB * NW
T0 = 10 * 25600
T1 = 6 * 25600
OFF1 = NS * T0


def _ebase(c, s):
    return jnp.where(c == 0, s * T0, OFF1 + s * T1)
NGP = 384

_mesh = plsc.VectorSubcoreMesh(
    core_axis_name="c", subcore_axis_name="s", num_cores=NC, num_subcores=NS)


def _wid():
    return lax.axis_index("s") * NC + lax.axis_index("c")


K1 = 12800
NB1 = TPB // K1


def _p1_body(d2_ref, batp_ref, ones_ref, zeros_ref,
             deg_out, ng_out, degT, ngT, oneb, didx0, didx1, bidx,
             sem0, sem1):
    c = lax.axis_index("c")
    s = lax.axis_index("s")
    wid = _wid()
    pltpu.sync_copy(ones_ref, oneb)
    pltpu.sync_copy(zeros_ref.at[pl.ds(s * NPT, NPT)],
                    degT.at[pl.ds(s * NPT, NPT)])

    @pl.when(jnp.logical_and(c == 0, s == 0))
    def _():
        pltpu.sync_copy(zeros_ref.at[pl.ds(0, NGP)], ngT)
    plsc.subcore_barrier()

    eb = _ebase(c, s)

    def eblk(j, carry):
        base = eb + 2 * j * K1
        pltpu.sync_copy(d2_ref.at[pl.ds(base, K1)], didx0)
        d0 = pltpu.async_copy(oneb.at[pl.ds(0, K1)], degT.at[didx0], sem0,
                              add=True)
        pltpu.sync_copy(d2_ref.at[pl.ds(base + K1, K1)], didx1)
        d1 = pltpu.async_copy(oneb.at[pl.ds(0, K1)], degT.at[didx1], sem1,
                              add=True)
        d0.wait()
        d1.wait()
        return carry

    @pl.when(c == 0)
    def _():
        lax.fori_loop(0, T0 // K1 // 2, eblk, 0)

    @pl.when(c != 0)
    def _():
        lax.fori_loop(0, T1 // K1 // 2, eblk, 0)

    @pl.when(c == 0)
    def _():
        pltpu.sync_copy(batp_ref.at[pl.ds(s * NPT, NPT)], bidx)
        pltpu.sync_copy(oneb.at[pl.ds(0, NPT)], ngT.at[bidx], add=True)
    plsc.subcore_barrier()
    pltpu.sync_copy(degT.at[pl.ds(s * NPT, NPT)],
                    deg_out.at[c, pl.ds(s * NPT, NPT)])

    @pl.when(jnp.logical_and(c == 0, s == 0))
    def _():
        pltpu.sync_copy(ngT, ng_out)


_p1 = pl.kernel(
    _p1_body,
    out_type=(jax.ShapeDtypeStruct((NC, NPAD), jnp.float32),
              jax.ShapeDtypeStruct((NGP,), jnp.float32)),
    mesh=_mesh,
    scratch_types=(
        pltpu.VMEM_SHARED((NPAD,), jnp.float32),
        pltpu.VMEM_SHARED((NGP,), jnp.float32),
        pltpu.VMEM((K1,), jnp.float32),
        pltpu.VMEM((K1,), jnp.int32),
        pltpu.VMEM((K1,), jnp.int32),
        pltpu.VMEM((NPT,), jnp.int32),
        pltpu.SemaphoreType.DMA,
        pltpu.SemaphoreType.DMA,
    ),
)


K2 = 6400
NB2 = TPB // K2
def _p2_body(s2_ref, d2_ref, v0_ref, v1_ref, v2_ref, z_ref,
             acc_out, accT0, accT1, accT2,
             sidx0, didx0, sidx1, didx1,
             b00, b01, b02, b10, b11, b12, semg0, semg1, sems0, sems1):
    c = lax.axis_index("c")
    s = lax.axis_index("s")
    wid = _wid()
    pltpu.sync_copy(z_ref.at[pl.ds(s * NPT, NPT)],
                    accT0.at[pl.ds(s * NPT, NPT)])
    pltpu.sync_copy(z_ref.at[pl.ds(s * NPT, NPT)],
                    accT1.at[pl.ds(s * NPT, NPT)])
    pltpu.sync_copy(z_ref.at[pl.ds(s * NPT, NPT)],
                    accT2.at[pl.ds(s * NPT, NPT)])
    plsc.subcore_barrier()

    eb = _ebase(c, s)

    def gather(i, sidx, didx, b0, b1, b2, sem):
        base = eb + i * K2
        pltpu.sync_copy(s2_ref.at[pl.ds(base, K2)], sidx)
        pltpu.sync_copy(d2_ref.at[pl.ds(base, K2)], didx)
        return (pltpu.async_copy(v0_ref.at[sidx], b0, sem),
                pltpu.async_copy(v1_ref.at[sidx], b1, sem),
                pltpu.async_copy(v2_ref.at[sidx], b2, sem))

    def scatter(didx, b0, b1, b2, sem):
        return (pltpu.async_copy(b0, accT0.at[didx], sem, add=True),
                pltpu.async_copy(b1, accT1.at[didx], sem, add=True),
                pltpu.async_copy(b2, accT2.at[didx], sem, add=True))

    def eblk(j, carry):
        g0 = gather(2 * j, sidx0, didx0, b00, b01, b02, semg0)
        g1 = gather(2 * j + 1, sidx1, didx1, b10, b11, b12, semg1)
        for d in g0:
            d.wait()
        s0 = scatter(didx0, b00, b01, b02, sems0)
        for d in g1:
            d.wait()
        s1 = scatter(didx1, b10, b11, b12, sems1)
        for d in s0:
            d.wait()
        for d in s1:
            d.wait()
        return carry

    @pl.when(c == 0)
    def _():
        lax.fori_loop(0, T0 // K2 // 2, eblk, 0)

    @pl.when(c != 0)
    def _():
        lax.fori_loop(0, T1 // K2 // 2, eblk, 0)
    plsc.subcore_barrier()
    pltpu.sync_copy(accT0.at[pl.ds(s * NPT, NPT)],
                    acc_out.at[c, 0, pl.ds(s * NPT, NPT)])
    pltpu.sync_copy(accT1.at[pl.ds(s * NPT, NPT)],
                    acc_out.at[c, 1, pl.ds(s * NPT, NPT)])
    pltpu.sync_copy(accT2.at[pl.ds(s * NPT, NPT)],
                    acc_out.at[c, 2, pl.ds(s * NPT, NPT)])


_p2 = pl.kernel(
    _p2_body,
    out_type=jax.ShapeDtypeStruct((NC, 3, NPAD), jnp.float32),
    mesh=_mesh,
    compiler_params=pltpu.CompilerParams(use_tc_tiling_on_sc=False),
    scratch_types=(
        pltpu.VMEM_SHARED((NPAD,), jnp.float32),
        pltpu.VMEM_SHARED((NPAD,), jnp.float32),
        pltpu.VMEM_SHARED((NPAD,), jnp.float32),
        pltpu.VMEM((K2,), jnp.int32),
        pltpu.VMEM((K2,), jnp.int32),
        pltpu.VMEM((K2,), jnp.int32),
        pltpu.VMEM((K2,), jnp.int32),
        pltpu.VMEM((K2,), jnp.float32),
        pltpu.VMEM((K2,), jnp.float32),
        pltpu.VMEM((K2,), jnp.float32),
        pltpu.VMEM((K2,), jnp.float32),
        pltpu.VMEM((K2,), jnp.float32),
        pltpu.VMEM((K2,), jnp.float32),
        pltpu.SemaphoreType.DMA,
        pltpu.SemaphoreType.DMA,
        pltpu.SemaphoreType.DMA,
        pltpu.SemaphoreType.DMA,
    ),
)


_UNROLL = 16
K3 = 2560
NB3 = TPB // K3


def _p3_body(s2_ref, d2_ref, u_ref, dinv_ref, batp_ref,
             p_out, sidx0, didx0, sidx1, didx1, ubuf0, ubuf1,
             wbuf0, wbuf1, gbuf0, gbuf1, P, sem0, sem1):
    c = lax.axis_index("c")
    s = lax.axis_index("s")
    wid = _wid()

    def zP(i, carry):
        P[i, :] = jnp.zeros((16,), jnp.float32)
        return carry

    lax.fori_loop(0, G + 1, zP, 0)

    eb = _ebase(c, s)

    def fetch(i, sidx, didx, ubuf, wbuf, gbuf, sem):
        base = eb + i * K3
        pltpu.sync_copy(s2_ref.at[pl.ds(base, K3)], sidx)
        pltpu.sync_copy(d2_ref.at[pl.ds(base, K3)], didx)
        du = pltpu.async_copy(u_ref.at[sidx], ubuf, sem)
        dw = pltpu.async_copy(dinv_ref.at[didx], wbuf, sem)
        dg = pltpu.async_copy(batp_ref.at[didx], gbuf, sem)
        return du, dw, dg

    def tec(ubuf, wbuf, gbuf):
        def edges(j, carry2):
            e0 = j * _UNROLL
            gv = gbuf[pl.ds(e0, 16)]
            wv = wbuf[pl.ds(e0, 16)]
            for t in range(_UNROLL):
                plsc.addupdate(P.at[gv[t]], ubuf[e0 + t, :] * wv[t])
            return carry2

        lax.fori_loop(0, K3 // _UNROLL, edges, 0)

    def pair(j, carry):
        d0 = fetch(2 * j, sidx0, didx0, ubuf0, wbuf0, gbuf0, sem0)
        d1 = fetch(2 * j + 1, sidx1, didx1, ubuf1, wbuf1, gbuf1, sem1)
        for d in d0:
            d.wait()
        tec(ubuf0, wbuf0, gbuf0)
        for d in d1:
            d.wait()
        tec(ubuf1, wbuf1, gbuf1)
        return carry

    @pl.when(c == 0)
    def _():
        lax.fori_loop(0, T0 // K3 // 2, pair, 0)

    @pl.when(c != 0)
    def _():
        lax.fori_loop(0, T1 // K3 // 2, pair, 0)
    pltpu.sync_copy(P, p_out.at[c, s])


_p3 = pl.kernel(
    _p3_body,
    out_type=jax.ShapeDtypeStruct((NC, NS, G + 1, 16), jnp.float32),
    mesh=_mesh,
    compiler_params=pltpu.CompilerParams(use_tc_tiling_on_sc=False),
    scratch_types=(
        pltpu.VMEM((K3,), jnp.int32),
        pltpu.VMEM((K3,), jnp.int32),
        pltpu.VMEM((K3,), jnp.int32),
        pltpu.VMEM((K3,), jnp.int32),
        pltpu.VMEM((K3, 16), jnp.float32),
        pltpu.VMEM((K3, 16), jnp.float32),
        pltpu.VMEM((K3,), jnp.float32),
        pltpu.VMEM((K3,), jnp.float32),
        pltpu.VMEM((K3,), jnp.int32),
        pltpu.VMEM((K3,), jnp.int32),
        pltpu.VMEM((G + 1, 16), jnp.float32),
        pltpu.SemaphoreType.DMA,
        pltpu.SemaphoreType.DMA,
    ),
)


_BN = 2048


def _ka_body(deg_ref, x4_ref, dinv_ref, v0_ref, v1_ref, v2_ref):
    d = deg_ref[...]
    dv = jnp.where(d > 0, lax.rsqrt(d), 0.0)
    dinv_ref[...] = dv
    v = dv * x4_ref[...]
    v0_ref[...] = v[:, 0:1]
    v1_ref[...] = v[:, 1:2]
    v2_ref[...] = v[:, 2:3]


def _kb_body(a0_ref, a1_ref, a2_ref, dinv_ref, w1_ref, b1_ref, u_ref):
    dv = dinv_ref[...]
    h = ((dv * a0_ref[...]) * w1_ref[0:1, :]
         + (dv * a1_ref[...]) * w1_ref[1:2, :]
         + (dv * a2_ref[...]) * w1_ref[2:3, :]
         + b1_ref[...])
    h = jnp.maximum(h, 0.0)
    u_ref[...] = dv * h


def _kc_body(p_ref, ng_ref, w2_ref, b2_ref, wl_ref, bl_ref, out_ref):
    pp = jnp.sum(p_ref[...], axis=0)[:G, :]
    ng = ng_ref[...][:G, :]
    pooled = jnp.dot(pp, w2_ref[...], preferred_element_type=jnp.float32)
    pooled = pooled + ng * b2_ref[...]
    lg = jnp.dot(pooled, wl_ref[...], preferred_element_type=jnp.float32)
    lg = lg + bl_ref[...]
    m = jnp.max(lg, axis=1, keepdims=True)
    ls = jnp.log(jnp.sum(jnp.exp(lg - m), axis=1, keepdims=True))
    out_ref[...] = lg - m - ls


def kernel(x, edge_index, batch, W1, b1, W2, b2, Wlin, blin):
    src = edge_index[0]
    dst = edge_index[1]
    e = src.shape[0]
    loop = jnp.arange(N, dtype=jnp.int32)
    padlen = EPAD - e - N
    sinkpad = jnp.full((padlen,), N, dtype=jnp.int32)
    s2 = jnp.concatenate([src, loop, sinkpad])
    d2 = jnp.concatenate([dst, loop, sinkpad])
    batp = jnp.concatenate(
        [batch, jnp.full((NPAD - N,), G, dtype=jnp.int32)])
    x4 = jnp.pad(x, ((0, NPAD - N), (0, 1)))
    ones_h = jnp.ones((K1,), jnp.float32)
    zeros_h = jnp.zeros((NPAD,), jnp.float32)

    deg2, ng = _p1(d2, batp, ones_h, zeros_h)
    degsum = (deg2[0] + deg2[1]).reshape(NPAD, 1)

    dinv, v0, v1, v2 = pl.pallas_call(
        _ka_body,
        grid=(NPAD // _BN,),
        in_specs=[pl.BlockSpec((_BN, 1), lambda i: (i, 0)),
                  pl.BlockSpec((_BN, 4), lambda i: (i, 0))],
        out_specs=[pl.BlockSpec((_BN, 1), lambda i: (i, 0))] * 4,
        out_shape=(jax.ShapeDtypeStruct((NPAD, 1), jnp.float32),) * 4,
    )(degsum, x4)

    acc2 = _p2(s2, d2, v0.reshape(NPAD), v1.reshape(NPAD),
               v2.reshape(NPAD), zeros_h)
    accsum = acc2[0] + acc2[1]

    u = pl.pallas_call(
        _kb_body,
        grid=(NPAD // _BN,),
        in_specs=[pl.BlockSpec((_BN, 1), lambda i: (i, 0))] * 4
        + [pl.BlockSpec((3, 16), lambda i: (0, 0)),
           pl.BlockSpec((1, 16), lambda i: (0, 0))],
        out_specs=pl.BlockSpec((_BN, 16), lambda i: (i, 0)),
        out_shape=jax.ShapeDtypeStruct((NPAD, 16), jnp.float32),
    )(accsum[0].reshape(NPAD, 1), accsum[1].reshape(NPAD, 1),
      accsum[2].reshape(NPAD, 1), dinv, W1, b1.reshape(1, 16))

    P = _p3(s2, d2, u, dinv.reshape(NPAD), batp)

    out = pl.pallas_call(
        _kc_body,
        out_shape=jax.ShapeDtypeStruct((G, 7), jnp.float32),
    )(P.reshape(NW, G + 1, 16), ng.reshape(NGP, 1),
      W2, b2.reshape(1, 16), Wlin, blin.reshape(1, 7))
    return out


# speedup vs baseline: 37.2761x; 1.0017x over previous
import jax
import jax.numpy as jnp
from jax import lax
from jax.experimental import pallas as pl
from jax.experimental.pallas import tpu as pltpu
from jax.experimental.pallas import tpu_sc as plsc

N = 100000
G = 256
NC, NS = 2, 16
NW = NC * NS
NPAD = 100352
NPT = NPAD // NS
TPB = 204800
EPAD = TPB * NW
T0 = 9 * 25600
T1 = 7 * 25600
OFF1 = NS * T0


def _ebase(c, s):
    return jnp.where(c == 0, s * T0, OFF1 + s * T1)
NGP = 384

_mesh = plsc.VectorSubcoreMesh(
    core_axis_name="c", subcore_axis_name="s", num_cores=NC, num_subcores=NS)


def _wid():
    return lax.axis_index("s") * NC + lax.axis_index("c")


K1 = 12800
NB1 = TPB // K1


def _p1_body(d2_ref, batp_ref, ones_ref, zeros_ref,
             deg_out, ng_out, degT, ngT, oneb, didx0, didx1, bidx,
             sem0, sem1):
    c = lax.axis_index("c")
    s = lax.axis_index("s")
    wid = _wid()
    pltpu.sync_copy(ones_ref, oneb)
    pltpu.sync_copy(zeros_ref.at[pl.ds(s * NPT, NPT)],
                    degT.at[pl.ds(s * NPT, NPT)])

    @pl.when(jnp.logical_and(c == 0, s == 0))
    def _():
        pltpu.sync_copy(zeros_ref.at[pl.ds(0, NGP)], ngT)
    plsc.subcore_barrier()

    eb = _ebase(c, s)

    def eblk(j, carry):
        base = eb + 2 * j * K1
        pltpu.sync_copy(d2_ref.at[pl.ds(base, K1)], didx0)
        d0 = pltpu.async_copy(oneb.at[pl.ds(0, K1)], degT.at[didx0], sem0,
                              add=True)
        pltpu.sync_copy(d2_ref.at[pl.ds(base + K1, K1)], didx1)
        d1 = pltpu.async_copy(oneb.at[pl.ds(0, K1)], degT.at[didx1], sem1,
                              add=True)
        d0.wait()
        d1.wait()
        return carry

    @pl.when(c == 0)
    def _():
        lax.fori_loop(0, T0 // K1 // 2, eblk, 0)

    @pl.when(c != 0)
    def _():
        lax.fori_loop(0, T1 // K1 // 2, eblk, 0)

    @pl.when(c == 0)
    def _():
        pltpu.sync_copy(batp_ref.at[pl.ds(s * NPT, NPT)], bidx)
        pltpu.sync_copy(oneb.at[pl.ds(0, NPT)], ngT.at[bidx], add=True)
    plsc.subcore_barrier()
    pltpu.sync_copy(degT.at[pl.ds(s * NPT, NPT)],
                    deg_out.at[c, pl.ds(s * NPT, NPT)])

    @pl.when(jnp.logical_and(c == 0, s == 0))
    def _():
        pltpu.sync_copy(ngT, ng_out)


_p1 = pl.kernel(
    _p1_body,
    out_type=(jax.ShapeDtypeStruct((NC, NPAD), jnp.float32),
              jax.ShapeDtypeStruct((NGP,), jnp.float32)),
    mesh=_mesh,
    scratch_types=(
        pltpu.VMEM_SHARED((NPAD,), jnp.float32),
        pltpu.VMEM_SHARED((NGP,), jnp.float32),
        pltpu.VMEM((K1,), jnp.float32),
        pltpu.VMEM((K1,), jnp.int32),
        pltpu.VMEM((K1,), jnp.int32),
        pltpu.VMEM((NPT,), jnp.int32),
        pltpu.SemaphoreType.DMA,
        pltpu.SemaphoreType.DMA,
    ),
)


K2 = 6400
NB2 = TPB // K2
def _p2_body(s2_ref, d2_ref, v0_ref, v1_ref, v2_ref, z_ref,
             acc_out, accT0, accT1, accT2,
             sidx0, didx0, sidx1, didx1,
             b00, b01, b02, b10, b11, b12, semg0, semg1, sems0, sems1):
    c = lax.axis_index("c")
    s = lax.axis_index("s")
    wid = _wid()
    pltpu.sync_copy(z_ref.at[pl.ds(s * NPT, NPT)],
                    accT0.at[pl.ds(s * NPT, NPT)])
    pltpu.sync_copy(z_ref.at[pl.ds(s * NPT, NPT)],
                    accT1.at[pl.ds(s * NPT, NPT)])
    pltpu.sync_copy(z_ref.at[pl.ds(s * NPT, NPT)],
                    accT2.at[pl.ds(s * NPT, NPT)])
    plsc.subcore_barrier()

    eb = _ebase(c, s)

    def gather(i, sidx, didx, b0, b1, b2, sem):
        base = eb + i * K2
        pltpu.sync_copy(s2_ref.at[pl.ds(base, K2)], sidx)
        pltpu.sync_copy(d2_ref.at[pl.ds(base, K2)], didx)
        return (pltpu.async_copy(v0_ref.at[sidx], b0, sem),
                pltpu.async_copy(v1_ref.at[sidx], b1, sem),
                pltpu.async_copy(v2_ref.at[sidx], b2, sem))

    def scatter(didx, b0, b1, b2, sem):
        return (pltpu.async_copy(b0, accT0.at[didx], sem, add=True),
                pltpu.async_copy(b1, accT1.at[didx], sem, add=True),
                pltpu.async_copy(b2, accT2.at[didx], sem, add=True))

    def eblk(j, carry):
        g0 = gather(2 * j, sidx0, didx0, b00, b01, b02, semg0)
        g1 = gather(2 * j + 1, sidx1, didx1, b10, b11, b12, semg1)
        for d in g0:
            d.wait()
        s0 = scatter(didx0, b00, b01, b02, sems0)
        for d in g1:
            d.wait()
        s1 = scatter(didx1, b10, b11, b12, sems1)
        for d in s0:
            d.wait()
        for d in s1:
            d.wait()
        return carry

    @pl.when(c == 0)
    def _():
        lax.fori_loop(0, T0 // K2 // 2, eblk, 0)

    @pl.when(c != 0)
    def _():
        lax.fori_loop(0, T1 // K2 // 2, eblk, 0)
    plsc.subcore_barrier()
    pltpu.sync_copy(accT0.at[pl.ds(s * NPT, NPT)],
                    acc_out.at[c, 0, pl.ds(s * NPT, NPT)])
    pltpu.sync_copy(accT1.at[pl.ds(s * NPT, NPT)],
                    acc_out.at[c, 1, pl.ds(s * NPT, NPT)])
    pltpu.sync_copy(accT2.at[pl.ds(s * NPT, NPT)],
                    acc_out.at[c, 2, pl.ds(s * NPT, NPT)])


_p2 = pl.kernel(
    _p2_body,
    out_type=jax.ShapeDtypeStruct((NC, 3, NPAD), jnp.float32),
    mesh=_mesh,
    compiler_params=pltpu.CompilerParams(use_tc_tiling_on_sc=False),
    scratch_types=(
        pltpu.VMEM_SHARED((NPAD,), jnp.float32),
        pltpu.VMEM_SHARED((NPAD,), jnp.float32),
        pltpu.VMEM_SHARED((NPAD,), jnp.float32),
        pltpu.VMEM((K2,), jnp.int32),
        pltpu.VMEM((K2,), jnp.int32),
        pltpu.VMEM((K2,), jnp.int32),
        pltpu.VMEM((K2,), jnp.int32),
        pltpu.VMEM((K2,), jnp.float32),
        pltpu.VMEM((K2,), jnp.float32),
        pltpu.VMEM((K2,), jnp.float32),
        pltpu.VMEM((K2,), jnp.float32),
        pltpu.VMEM((K2,), jnp.float32),
        pltpu.VMEM((K2,), jnp.float32),
        pltpu.SemaphoreType.DMA,
        pltpu.SemaphoreType.DMA,
        pltpu.SemaphoreType.DMA,
        pltpu.SemaphoreType.DMA,
    ),
)


_UNROLL = 16
K3 = 2560
NB3 = TPB // K3


def _p3_body(s2_ref, d2_ref, u_ref, dinv_ref, batp_ref,
             p_out, sidx0, didx0, sidx1, didx1, ubuf0, ubuf1,
             wbuf0, wbuf1, gbuf0, gbuf1, P, sem0, sem1):
    c = lax.axis_index("c")
    s = lax.axis_index("s")
    wid = _wid()

    def zP(i, carry):
        P[i, :] = jnp.zeros((16,), jnp.float32)
        return carry

    lax.fori_loop(0, G + 1, zP, 0)

    eb = _ebase(c, s)

    def fetch(i, sidx, didx, ubuf, wbuf, gbuf, sem):
        base = eb + i * K3
        pltpu.sync_copy(s2_ref.at[pl.ds(base, K3)], sidx)
        pltpu.sync_copy(d2_ref.at[pl.ds(base, K3)], didx)
        du = pltpu.async_copy(u_ref.at[sidx], ubuf, sem)
        dw = pltpu.async_copy(dinv_ref.at[didx], wbuf, sem)
        dg = pltpu.async_copy(batp_ref.at[didx], gbuf, sem)
        return du, dw, dg

    def tec(ubuf, wbuf, gbuf):
        def edges(j, carry2):
            e0 = j * _UNROLL
            gv = gbuf[pl.ds(e0, 16)]
            wv = wbuf[pl.ds(e0, 16)]
            for t in range(_UNROLL):
                plsc.addupdate(P.at[gv[t]], ubuf[e0 + t, :] * wv[t])
            return carry2

        lax.fori_loop(0, K3 // _UNROLL, edges, 0)

    def pair(j, carry):
        d0 = fetch(2 * j, sidx0, didx0, ubuf0, wbuf0, gbuf0, sem0)
        d1 = fetch(2 * j + 1, sidx1, didx1, ubuf1, wbuf1, gbuf1, sem1)
        for d in d0:
            d.wait()
        tec(ubuf0, wbuf0, gbuf0)
        for d in d1:
            d.wait()
        tec(ubuf1, wbuf1, gbuf1)
        return carry

    @pl.when(c == 0)
    def _():
        lax.fori_loop(0, T0 // K3 // 2, pair, 0)

    @pl.when(c != 0)
    def _():
        lax.fori_loop(0, T1 // K3 // 2, pair, 0)
    pltpu.sync_copy(P, p_out.at[c, s])


_p3 = pl.kernel(
    _p3_body,
    out_type=jax.ShapeDtypeStruct((NC, NS, G + 1, 16), jnp.float32),
    mesh=_mesh,
    compiler_params=pltpu.CompilerParams(use_tc_tiling_on_sc=False),
    scratch_types=(
        pltpu.VMEM((K3,), jnp.int32),
        pltpu.VMEM((K3,), jnp.int32),
        pltpu.VMEM((K3,), jnp.int32),
        pltpu.VMEM((K3,), jnp.int32),
        pltpu.VMEM((K3, 16), jnp.float32),
        pltpu.VMEM((K3, 16), jnp.float32),
        pltpu.VMEM((K3,), jnp.float32),
        pltpu.VMEM((K3,), jnp.float32),
        pltpu.VMEM((K3,), jnp.int32),
        pltpu.VMEM((K3,), jnp.int32),
        pltpu.VMEM((G + 1, 16), jnp.float32),
        pltpu.SemaphoreType.DMA,
        pltpu.SemaphoreType.DMA,
    ),
)


_BN = 2048


def _ka_body(deg_ref, x4_ref, dinv_ref, v0_ref, v1_ref, v2_ref):
    d = deg_ref[...]
    dv = jnp.where(d > 0, lax.rsqrt(d), 0.0)
    dinv_ref[...] = dv
    v = dv * x4_ref[...]
    v0_ref[...] = v[:, 0:1]
    v1_ref[...] = v[:, 1:2]
    v2_ref[...] = v[:, 2:3]


def _kb_body(a0_ref, a1_ref, a2_ref, dinv_ref, w1_ref, b1_ref, u_ref):
    dv = dinv_ref[...]
    h = ((dv * a0_ref[...]) * w1_ref[0:1, :]
         + (dv * a1_ref[...]) * w1_ref[1:2, :]
         + (dv * a2_ref[...]) * w1_ref[2:3, :]
         + b1_ref[...])
    h = jnp.maximum(h, 0.0)
    u_ref[...] = dv * h


def _kc_body(p_ref, ng_ref, w2_ref, b2_ref, wl_ref, bl_ref, out_ref):
    pp = jnp.sum(p_ref[...], axis=0)[:G, :]
    ng = ng_ref[...][:G, :]
    pooled = jnp.dot(pp, w2_ref[...], preferred_element_type=jnp.float32)
    pooled = pooled + ng * b2_ref[...]
    lg = jnp.dot(pooled, wl_ref[...], preferred_element_type=jnp.float32)
    lg = lg + bl_ref[...]
    m = jnp.max(lg, axis=1, keepdims=True)
    ls = jnp.log(jnp.sum(jnp.exp(lg - m), axis=1, keepdims=True))
    out_ref[...] = lg - m - ls


def kernel(x, edge_index, batch, W1, b1, W2, b2, Wlin, blin):
    src = edge_index[0]
    dst = edge_index[1]
    e = src.shape[0]
    loop = jnp.arange(N, dtype=jnp.int32)
    padlen = EPAD - e - N
    sinkpad = jnp.full((padlen,), N, dtype=jnp.int32)
    s2 = jnp.concatenate([src, loop, sinkpad])
    d2 = jnp.concatenate([dst, loop, sinkpad])
    batp = jnp.concatenate(
        [batch, jnp.full((NPAD - N,), G, dtype=jnp.int32)])
    x4 = jnp.pad(x, ((0, NPAD - N), (0, 1)))
    ones_h = jnp.ones((K1,), jnp.float32)
    zeros_h = jnp.zeros((NPAD,), jnp.float32)

    deg2, ng = _p1(d2, batp, ones_h, zeros_h)
    degsum = (deg2[0] + deg2[1]).reshape(NPAD, 1)

    dinv, v0, v1, v2 = pl.pallas_call(
        _ka_body,
        grid=(NPAD // _BN,),
        in_specs=[pl.BlockSpec((_BN, 1), lambda i: (i, 0)),
                  pl.BlockSpec((_BN, 4), lambda i: (i, 0))],
        out_specs=[pl.BlockSpec((_BN, 1), lambda i: (i, 0))] * 4,
        out_shape=(jax.ShapeDtypeStruct((NPAD, 1), jnp.float32),) * 4,
    )(degsum, x4)

    acc2 = _p2(s2, d2, v0.reshape(NPAD), v1.reshape(NPAD),
               v2.reshape(NPAD), zeros_h)
    accsum = acc2[0] + acc2[1]

    u = pl.pallas_call(
        _kb_body,
        grid=(NPAD // _BN,),
        in_specs=[pl.BlockSpec((_BN, 1), lambda i: (i, 0))] * 4
        + [pl.BlockSpec((3, 16), lambda i: (0, 0)),
           pl.BlockSpec((1, 16), lambda i: (0, 0))],
        out_specs=pl.BlockSpec((_BN, 16), lambda i: (i, 0)),
        out_shape=jax.ShapeDtypeStruct((NPAD, 16), jnp.float32),
    )(accsum[0].reshape(NPAD, 1), accsum[1].reshape(NPAD, 1),
      accsum[2].reshape(NPAD, 1), dinv, W1, b1.reshape(1, 16))

    P = _p3(s2, d2, u, dinv.reshape(NPAD), batp)

    out = pl.pallas_call(
        _kc_body,
        out_shape=jax.ShapeDtypeStruct((G, 7), jnp.float32),
    )(P.reshape(NW, G + 1, 16), ng.reshape(NGP, 1),
      W2, b2.reshape(1, 16), Wlin, blin.reshape(1, 7))
    return out
